# baseline (speedup 1.0000x reference)
import functools

import jax
import jax.numpy as jnp
from jax import lax
from jax.experimental import pallas as pl
from jax.experimental.pallas import tpu as pltpu
from jax.experimental.pallas import tpu_sc as plsc

_D = 64
_B_TOTAL = 16384 * 50
_NC = 2
_NS = 16
_NW = _NC * _NS
_BPW = _B_TOTAL // _NW
_G = 128
_GROUPS = _BPW // _G


def _body(table_hbm, ids_hbm, out_hbm, idx_v, rows_v, sem):
    wid = lax.axis_index("s") * _NC + lax.axis_index("c")
    grp_base = wid * _GROUPS
    row_base = wid * _BPW

    pltpu.sync_copy(ids_hbm.at[pl.ds(grp_base, _GROUPS)], idx_v)

    def step(j, _):
        pltpu.async_copy(table_hbm.at[idx_v.at[j]], rows_v, sem).wait()
        pltpu.sync_copy(rows_v, out_hbm.at[pl.ds(row_base + j * _G, _G)])
        return 0

    lax.fori_loop(0, _GROUPS, step, 0)


@jax.jit
def _gather(table, ids2d):
    mesh = plsc.VectorSubcoreMesh(core_axis_name="c", subcore_axis_name="s")
    return pl.kernel(
        _body,
        mesh=mesh,
        compiler_params=pltpu.CompilerParams(use_tc_tiling_on_sc=False),
        out_type=jax.ShapeDtypeStruct((_B_TOTAL, _D), jnp.float32),
        scratch_types=[
            pltpu.VMEM((_GROUPS, _G), jnp.int32),
            pltpu.VMEM((_G, _D), jnp.float32),
            pltpu.SemaphoreType.DMA,
        ],
    )(table, ids2d)


def kernel(input_ids, table):
    ids2d = input_ids.reshape(_NW * _GROUPS, _G).astype(jnp.int32)
    out = _gather(table, ids2d)
    return out.reshape(-1, 1, _D)


# speedup vs baseline: 1.1147x; 1.1147x over previous
import functools

import jax
import jax.numpy as jnp
from jax import lax
from jax.experimental import pallas as pl
from jax.experimental.pallas import tpu as pltpu
from jax.experimental.pallas import tpu_sc as plsc

_D = 64
_B_TOTAL = 16384 * 50
_NC = 2
_NS = 16
_NW = _NC * _NS
_BPW = _B_TOTAL // _NW
_G = 128
_GROUPS = _BPW // _G


_K = 4
_CH = _K * _G
_NCHUNK = _GROUPS // _K


def _body(table_hbm, ids_hbm, out_hbm, idx_v, buf0, buf1, sem0, sem1):
    wid = lax.axis_index("s") * _NC + lax.axis_index("c")
    grp_base = wid * _GROUPS
    row_base = wid * _BPW

    pltpu.sync_copy(ids_hbm.at[pl.ds(grp_base, _GROUPS)], idx_v)

    def fire(chunk, buf, sem):
        for k in range(_K):
            pltpu.async_copy(table_hbm.at[idx_v.at[chunk * _K + k]],
                             buf.at[pl.ds(k * _G, _G)], sem)

    def drain(buf, sem):
        pltpu.make_async_copy(table_hbm.at[pl.ds(0, _CH)], buf, sem).wait()

    fire(0, buf0, sem0)

    def step(i, _):
        c_a = 2 * i
        c_b = 2 * i + 1
        c_c = 2 * i + 2
        fire(c_b, buf1, sem1)
        drain(buf0, sem0)
        pltpu.sync_copy(buf0, out_hbm.at[pl.ds(row_base + c_a * _CH, _CH)])

        @pl.when(c_c < _NCHUNK)
        def _():
            fire(c_c, buf0, sem0)

        drain(buf1, sem1)
        pltpu.sync_copy(buf1, out_hbm.at[pl.ds(row_base + c_b * _CH, _CH)])
        return 0

    lax.fori_loop(0, _NCHUNK // 2, step, 0)


@jax.jit
def _gather(table, ids2d):
    mesh = plsc.VectorSubcoreMesh(core_axis_name="c", subcore_axis_name="s")
    return pl.kernel(
        _body,
        mesh=mesh,
        compiler_params=pltpu.CompilerParams(use_tc_tiling_on_sc=False),
        out_type=jax.ShapeDtypeStruct((_B_TOTAL, _D), jnp.float32),
        scratch_types=[
            pltpu.VMEM((_GROUPS, _G), jnp.int32),
            pltpu.VMEM((_CH, _D), jnp.float32),
            pltpu.VMEM((_CH, _D), jnp.float32),
            pltpu.SemaphoreType.DMA,
            pltpu.SemaphoreType.DMA,
        ],
    )(table, ids2d)


def kernel(input_ids, table):
    ids2d = input_ids.reshape(_NW * _GROUPS, _G).astype(jnp.int32)
    out = _gather(table, ids2d)
    return out.reshape(-1, 1, _D)
